# Initial kernel scaffold; baseline (speedup 1.0000x reference)
import functools

import jax
import jax.numpy as jnp
from jax import lax
from jax.experimental import pallas as pl
from jax.experimental.pallas import tpu as pltpu
from jax.experimental.pallas import tpu_sc as plsc

_N = 10000
_E = 320000
_D_IN = 128
_HID = 16
_HEADS = 8
_D1 = _HEADS * _HID
_D_OUT = 64

_NCORES = 2
_NSUB = 16
_NW = _NCORES * _NSUB
_ROWS_PER_TILE = _N // _NSUB

_B = 80
_EPW = _E // _NW
_CHUNKS = _EPW // _B

_W1COLS = 144
_W2COLS = 80


def _leaky(v):
    return jnp.where(v >= 0, v, 0.2 * v)


def _sc_edges1(h_hbm, att_hbm, ei_hbm, z_hbm, acc_out,
               src_v, dst_v, rows_v, as_v, ad_v, msg_v, acc_sh, sem):
    cid = lax.axis_index("c")
    sid = lax.axis_index("s")
    wid = cid * _NSUB + sid
    r0 = sid * _ROWS_PER_TILE
    pltpu.sync_copy(z_hbm.at[pl.ds(r0, _ROWS_PER_TILE), :],
                    acc_sh.at[pl.ds(r0, _ROWS_PER_TILE), :])
    zero16 = jnp.zeros((16,), jnp.float32)

    def _zpad(k, c):
        msg_v[k, pl.ds(128, 16)] = zero16
        return c

    lax.fori_loop(0, _B, _zpad, 0)
    plsc.subcore_barrier()

    base = wid * _EPW
    iota = lax.iota(jnp.int32, 16)

    def _chunk(c, carry):
        off = base + c * _B
        pltpu.sync_copy(ei_hbm.at[0, pl.ds(off, _B)], src_v)
        pltpu.sync_copy(ei_hbm.at[1, pl.ds(off, _B)], dst_v)
        pltpu.async_copy(h_hbm.at[src_v], rows_v, sem).wait()
        pltpu.async_copy(att_hbm.at[src_v], as_v, sem).wait()
        pltpu.async_copy(att_hbm.at[dst_v], ad_v, sem).wait()
        for g in range(_B // 16):
            ridx = iota + g * 16
            for h in range(_HEADS):
                a_s = plsc.load_gather(as_v, [ridx, jnp.full((16,), h, jnp.int32)])
                a_d = plsc.load_gather(ad_v, [ridx, jnp.full((16,), 8 + h, jnp.int32)])
                w = jnp.exp(_leaky(a_s + a_d))
                plsc.store_scatter(msg_v, [ridx, jnp.full((16,), 128 + h, jnp.int32)], w)

        def _edge(k, cc):
            kk = jnp.full((16,), k, jnp.int32)
            for h in range(_HEADS):
                wv = plsc.load_gather(msg_v, [kk, jnp.full((16,), 128 + h, jnp.int32)])
                msg_v[k, pl.ds(h * 16, 16)] = rows_v[k, pl.ds(h * 16, 16)] * wv
            return cc

        lax.fori_loop(0, _B, _edge, 0)
        pltpu.sync_copy(msg_v, acc_sh.at[dst_v], add=True)
        return carry

    lax.fori_loop(0, _CHUNKS, _chunk, 0)
    plsc.subcore_barrier()
    pltpu.sync_copy(acc_sh.at[pl.ds(r0, _ROWS_PER_TILE), :],
                    acc_out.at[cid, pl.ds(r0, _ROWS_PER_TILE), :])


def _sc_edges2(h_hbm, att_hbm, ei_hbm, z_hbm, acc_out,
               src_v, dst_v, rows_v, as_v, ad_v, msg_v, acc_sh, sem):
    cid = lax.axis_index("c")
    sid = lax.axis_index("s")
    wid = cid * _NSUB + sid
    r0 = sid * _ROWS_PER_TILE
    pltpu.sync_copy(z_hbm.at[pl.ds(r0, _ROWS_PER_TILE), :],
                    acc_sh.at[pl.ds(r0, _ROWS_PER_TILE), :])
    zero16 = jnp.zeros((16,), jnp.float32)

    def _zpad(k, c):
        msg_v[k, pl.ds(64, 16)] = zero16
        return c

    lax.fori_loop(0, _B, _zpad, 0)
    plsc.subcore_barrier()

    base = wid * _EPW
    iota = lax.iota(jnp.int32, 16)

    def _chunk(c, carry):
        off = base + c * _B
        pltpu.sync_copy(ei_hbm.at[0, pl.ds(off, _B)], src_v)
        pltpu.sync_copy(ei_hbm.at[1, pl.ds(off, _B)], dst_v)
        pltpu.async_copy(h_hbm.at[src_v], rows_v, sem).wait()
        pltpu.async_copy(att_hbm.at[src_v], as_v, sem).wait()
        pltpu.async_copy(att_hbm.at[dst_v], ad_v, sem).wait()
        for g in range(_B // 16):
            ridx = iota + g * 16
            a_s = plsc.load_gather(as_v, [ridx, jnp.zeros((16,), jnp.int32)])
            a_d = plsc.load_gather(ad_v, [ridx, jnp.ones((16,), jnp.int32)])
            w = jnp.exp(_leaky(a_s + a_d))
            plsc.store_scatter(msg_v, [ridx, jnp.full((16,), 64, jnp.int32)], w)

        def _edge(k, cc):
            kk = jnp.full((16,), k, jnp.int32)
            wv = plsc.load_gather(msg_v, [kk, jnp.full((16,), 64, jnp.int32)])
            for h in range(_D_OUT // 16):
                msg_v[k, pl.ds(h * 16, 16)] = rows_v[k, pl.ds(h * 16, 16)] * wv
            return cc

        lax.fori_loop(0, _B, _edge, 0)
        pltpu.sync_copy(msg_v, acc_sh.at[dst_v], add=True)
        return carry

    lax.fori_loop(0, _CHUNKS, _chunk, 0)
    plsc.subcore_barrier()
    pltpu.sync_copy(acc_sh.at[pl.ds(r0, _ROWS_PER_TILE), :],
                    acc_out.at[cid, pl.ds(r0, _ROWS_PER_TILE), :])


def _make_sc_call(body, d_row, n_cols):
    mesh = plsc.VectorSubcoreMesh(core_axis_name="c", subcore_axis_name="s")
    return pl.kernel(
        body,
        out_type=jax.ShapeDtypeStruct((_NCORES, _N, n_cols), jnp.float32),
        mesh=mesh,
        scratch_types=[
            pltpu.VMEM((_B,), jnp.int32),
            pltpu.VMEM((_B,), jnp.int32),
            pltpu.VMEM((_B, d_row), jnp.float32),
            pltpu.VMEM((_B, 16), jnp.float32),
            pltpu.VMEM((_B, 16), jnp.float32),
            pltpu.VMEM((_B, n_cols), jnp.float32),
            pltpu.VMEM_SHARED((_N, n_cols), jnp.float32),
            pltpu.SemaphoreType.DMA,
        ],
    )


def _tc_stage_a(x_ref, w1_ref, aproj_ref, h_out, att_out):
    h = jnp.dot(x_ref[...], w1_ref[...], preferred_element_type=jnp.float32)
    h_out[...] = h
    att_out[...] = jnp.dot(h, aproj_ref[...], preferred_element_type=jnp.float32)


def _tc_stage_b(acc_ref, h1_ref, att1_ref, b1_ref, w2_ref, r_ref, a2_ref,
                h2_out, att2_out):
    att = att1_ref[...]
    eself = att[:, 0:8] + att[:, 8:16]
    wself = jnp.exp(_leaky(eself))
    num = acc_ref[0, :, 0:128] + acc_ref[1, :, 0:128]
    den = acc_ref[0, :, 128:136] + acc_ref[1, :, 128:136] + wself
    wexp = jnp.dot(wself, r_ref[...], preferred_element_type=jnp.float32)
    dexp = jnp.dot(den, r_ref[...], preferred_element_type=jnp.float32)
    num = num + h1_ref[...] * wexp
    z = num / dexp + b1_ref[...]
    z = jnp.where(z > 0, z, jnp.exp(jnp.minimum(z, 0.0)) - 1.0)
    h2 = jnp.dot(z, w2_ref[...], preferred_element_type=jnp.float32)
    h2_out[...] = h2
    att2_out[...] = jnp.dot(h2, a2_ref[...], preferred_element_type=jnp.float32)


def _tc_stage_c(acc_ref, h2_ref, att2_ref, b2_ref, out_ref):
    att = att2_ref[...]
    wself = jnp.exp(_leaky(att[:, 0:1] + att[:, 1:2]))
    num = acc_ref[0, :, 0:64] + acc_ref[1, :, 0:64] + h2_ref[...] * wself
    den = acc_ref[0, :, 64:65] + acc_ref[1, :, 64:65] + wself
    o = num / den + b2_ref[...]
    m = jnp.max(o, axis=1, keepdims=True)
    lse = jnp.log(jnp.sum(jnp.exp(o - m), axis=1, keepdims=True)) + m
    out_ref[...] = o - lse


def kernel(x, edge_index, W1, att_src1, att_dst1, b1, W2, att_src2, att_dst2, b2):
    f32 = jnp.float32
    eye_h = jnp.eye(_HEADS, dtype=f32)
    t_src = (eye_h[:, None, :] * att_src1.astype(f32).T[:, None, :].transpose(1, 0, 2)).reshape(_D1, _HEADS)
    t_dst = (eye_h[:, None, :] * att_dst1.astype(f32).T[:, None, :].transpose(1, 0, 2)).reshape(_D1, _HEADS)
    aproj = jnp.concatenate([t_src, t_dst], axis=1)
    r_mat = jnp.kron(jnp.eye(_HEADS, dtype=f32), jnp.ones((1, _HID), f32))
    a2 = jnp.concatenate([att_src2.astype(f32).T, att_dst2.astype(f32).T,
                          jnp.zeros((_D_OUT, 14), f32)], axis=1)
    ei = edge_index.astype(jnp.int32)
    zeros1 = jnp.zeros((_N, _W1COLS), f32)
    zeros2 = jnp.zeros((_N, _W2COLS), f32)

    h1, att1 = pl.pallas_call(
        _tc_stage_a,
        out_shape=[jax.ShapeDtypeStruct((_N, _D1), f32),
                   jax.ShapeDtypeStruct((_N, 16), f32)],
    )(x.astype(f32), W1.astype(f32), aproj)

    acc1 = _make_sc_call(_sc_edges1, _D1, _W1COLS)(h1, att1, ei, zeros1)

    h2, att2 = pl.pallas_call(
        _tc_stage_b,
        out_shape=[jax.ShapeDtypeStruct((_N, _D_OUT), f32),
                   jax.ShapeDtypeStruct((_N, 16), f32)],
    )(acc1, h1, att1, b1.astype(f32).reshape(1, _D1), W2.astype(f32), r_mat, a2)

    acc2 = _make_sc_call(_sc_edges2, _D_OUT, _W2COLS)(h2, att2, ei, zeros2)

    out = pl.pallas_call(
        _tc_stage_c,
        out_shape=jax.ShapeDtypeStruct((_N, _D_OUT), f32),
    )(acc2, h2, att2, b2.astype(f32).reshape(1, _D_OUT))
    return out


# speedup vs baseline: 35.8967x; 35.8967x over previous
import jax
import jax.numpy as jnp
from jax import lax
from jax.experimental import pallas as pl
from jax.experimental.pallas import tpu as pltpu
from jax.experimental.pallas import tpu_sc as plsc

_N = 10000
_E = 320000
_D_IN = 128
_HID = 16
_HEADS = 8
_D1 = _HEADS * _HID
_D_OUT = 64

_NCORES = 2
_NSUB = 16
_NPAD = 10240
_ROWS_PER_TILE = _NPAD // _NSUB

_B = 80


def _leaky(v):
    return jnp.where(v >= 0, v, 0.2 * v)


def _sc_edges1(ts_hbm, td_hbm, esrc_hbm, edst_hbm, z_hbm, acc_out,
               src_v, dst_v, dstg_v, srow_v, drow_v, msg_v, acc_sh, sem):
    cid = lax.axis_index("c")
    sid = lax.axis_index("s")
    r0 = sid * _ROWS_PER_TILE
    pltpu.sync_copy(z_hbm.at[pl.ds(r0, _ROWS_PER_TILE), :],
                    acc_sh.at[pl.ds(r0, _ROWS_PER_TILE), :])
    plsc.subcore_barrier()

    epw = _E // _NSUB
    chunks = epw // _B
    base = sid * epw
    lane = lax.iota(jnp.int32, 16)
    goff = jnp.full((16,), cid * _N, jnp.int32)

    def _chunk(c, carry):
        off = base + c * _B
        pltpu.sync_copy(esrc_hbm.at[pl.ds(off, _B)], src_v)
        pltpu.sync_copy(edst_hbm.at[pl.ds(off, _B)], dst_v)
        for g in range(_B // 16):
            sl = pl.ds(g * 16, 16)
            src_v[sl] = src_v[sl] + goff
            dstg_v[sl] = dst_v[sl] + goff
        pltpu.async_copy(ts_hbm.at[src_v], srow_v, sem).wait()
        pltpu.async_copy(td_hbm.at[dstg_v], drow_v, sem).wait()

        def _edge(k, cc):
            w16 = jnp.exp(_leaky(srow_v[k, pl.ds(64, 16)] + drow_v[k, pl.ds(0, 16)]))
            msg_v[k, pl.ds(64, 16)] = jnp.where(lane < 4, w16, 0.0)
            for h in range(4):
                wv = jnp.full((16,), w16[h], jnp.float32)
                msg_v[k, pl.ds(h * 16, 16)] = srow_v[k, pl.ds(h * 16, 16)] * wv
            return cc

        lax.fori_loop(0, _B, _edge, 0)
        pltpu.sync_copy(msg_v, acc_sh.at[dst_v], add=True)
        return carry

    lax.fori_loop(0, chunks, _chunk, 0)
    plsc.subcore_barrier()
    pltpu.sync_copy(acc_sh.at[pl.ds(r0, _ROWS_PER_TILE), :],
                    acc_out.at[cid, pl.ds(r0, _ROWS_PER_TILE), :])


def _sc_edges2(ts_hbm, td_hbm, esrc_hbm, edst_hbm, z_hbm, acc_out,
               src_v, dst_v, srow_v, drow_v, msg_v, acc_sh, sem):
    cid = lax.axis_index("c")
    sid = lax.axis_index("s")
    r0 = sid * _ROWS_PER_TILE
    pltpu.sync_copy(z_hbm.at[pl.ds(r0, _ROWS_PER_TILE), :],
                    acc_sh.at[pl.ds(r0, _ROWS_PER_TILE), :])
    plsc.subcore_barrier()

    epw = _E // (_NCORES * _NSUB)
    chunks = epw // _B
    base = (cid * _NSUB + sid) * epw
    lane = lax.iota(jnp.int32, 16)

    def _chunk(c, carry):
        off = base + c * _B
        pltpu.sync_copy(esrc_hbm.at[pl.ds(off, _B)], src_v)
        pltpu.sync_copy(edst_hbm.at[pl.ds(off, _B)], dst_v)
        pltpu.async_copy(ts_hbm.at[src_v], srow_v, sem).wait()
        pltpu.async_copy(td_hbm.at[dst_v], drow_v, sem).wait()

        def _edge(k, cc):
            w16 = jnp.exp(_leaky(srow_v[k, pl.ds(64, 16)] + drow_v[k, pl.ds(0, 16)]))
            msg_v[k, pl.ds(64, 16)] = jnp.where(lane < 1, w16, 0.0)
            wv = jnp.full((16,), w16[0], jnp.float32)
            for h in range(_D_OUT // 16):
                msg_v[k, pl.ds(h * 16, 16)] = srow_v[k, pl.ds(h * 16, 16)] * wv
            return cc

        lax.fori_loop(0, _B, _edge, 0)
        pltpu.sync_copy(msg_v, acc_sh.at[dst_v], add=True)
        return carry

    lax.fori_loop(0, chunks, _chunk, 0)
    plsc.subcore_barrier()
    pltpu.sync_copy(acc_sh.at[pl.ds(r0, _ROWS_PER_TILE), :],
                    acc_out.at[cid, pl.ds(r0, _ROWS_PER_TILE), :])


def _sc_call1():
    mesh = plsc.VectorSubcoreMesh(core_axis_name="c", subcore_axis_name="s")
    return pl.kernel(
        _sc_edges1,
        out_type=jax.ShapeDtypeStruct((_NCORES, _NPAD, 128), jnp.float32),
        mesh=mesh,
        scratch_types=[
            pltpu.VMEM((_B,), jnp.int32),
            pltpu.VMEM((_B,), jnp.int32),
            pltpu.VMEM((_B,), jnp.int32),
            pltpu.VMEM((_B, 128), jnp.float32),
            pltpu.VMEM((_B, 128), jnp.float32),
            pltpu.VMEM((_B, 128), jnp.float32),
            pltpu.VMEM_SHARED((_NPAD, 128), jnp.float32),
            pltpu.SemaphoreType.DMA,
        ],
    )


def _sc_call2():
    mesh = plsc.VectorSubcoreMesh(core_axis_name="c", subcore_axis_name="s")
    return pl.kernel(
        _sc_edges2,
        out_type=jax.ShapeDtypeStruct((_NCORES, _NPAD, 128), jnp.float32),
        mesh=mesh,
        scratch_types=[
            pltpu.VMEM((_B,), jnp.int32),
            pltpu.VMEM((_B,), jnp.int32),
            pltpu.VMEM((_B, 128), jnp.float32),
            pltpu.VMEM((_B, 128), jnp.float32),
            pltpu.VMEM((_B, 128), jnp.float32),
            pltpu.VMEM_SHARED((_NPAD, 128), jnp.float32),
            pltpu.SemaphoreType.DMA,
        ],
    )


def _tc_stage_a(x_ref, w1_ref, aproj_ref, ts_out, td_out):
    h = jnp.dot(x_ref[...], w1_ref[...], preferred_element_type=jnp.float32)
    att = jnp.dot(h, aproj_ref[...], preferred_element_type=jnp.float32)
    n = h.shape[0]
    z56 = jnp.zeros((n, 56), jnp.float32)
    z120 = jnp.zeros((n, 120), jnp.float32)
    ts_out[0:10000, :] = jnp.concatenate(
        [h[:, 0:64], att[:, 0:4], att[:, 8:12], z56], axis=1)
    ts_out[10000:20000, :] = jnp.concatenate(
        [h[:, 64:128], att[:, 4:8], att[:, 12:16], z56], axis=1)
    td_out[0:10000, :] = jnp.concatenate([att[:, 8:12], jnp.zeros((n, 12), jnp.float32), z120[:, 0:112]], axis=1)
    td_out[10000:20000, :] = jnp.concatenate([att[:, 12:16], jnp.zeros((n, 12), jnp.float32), z120[:, 0:112]], axis=1)


def _tc_stage_b(acc_ref, ts1_ref, b1_ref, w2_ref, r_ref, a2_ref, ts_out, td_out):
    h1 = jnp.concatenate([ts1_ref[0:10000, 0:64], ts1_ref[10000:20000, 0:64]], axis=1)
    as1 = jnp.concatenate([ts1_ref[0:10000, 64:68], ts1_ref[10000:20000, 64:68]], axis=1)
    ad1 = jnp.concatenate([ts1_ref[0:10000, 68:72], ts1_ref[10000:20000, 68:72]], axis=1)
    wself = jnp.exp(_leaky(as1 + ad1))
    num = jnp.concatenate([acc_ref[0, 0:10000, 0:64], acc_ref[1, 0:10000, 0:64]], axis=1)
    den = jnp.concatenate([acc_ref[0, 0:10000, 64:68], acc_ref[1, 0:10000, 64:68]], axis=1)
    den = den + wself
    wexp = jnp.dot(wself, r_ref[...], preferred_element_type=jnp.float32)
    dexp = jnp.dot(den, r_ref[...], preferred_element_type=jnp.float32)
    num = num + h1 * wexp
    z = num / dexp + b1_ref[...]
    z = jnp.where(z > 0, z, jnp.exp(jnp.minimum(z, 0.0)) - 1.0)
    h2 = jnp.dot(z, w2_ref[...], preferred_element_type=jnp.float32)
    att2 = jnp.dot(h2, a2_ref[...], preferred_element_type=jnp.float32)
    n = h2.shape[0]
    ts_out[...] = jnp.concatenate([h2, att2, jnp.zeros((n, 62), jnp.float32)], axis=1)
    td_out[...] = jnp.concatenate([att2[:, 1:2], jnp.zeros((n, 127), jnp.float32)], axis=1)


def _tc_stage_c(acc_ref, ts2_ref, b2_ref, out_ref):
    h2 = ts2_ref[:, 0:64]
    wself = jnp.exp(_leaky(ts2_ref[:, 64:65] + ts2_ref[:, 65:66]))
    num = acc_ref[0, 0:10000, 0:64] + acc_ref[1, 0:10000, 0:64] + h2 * wself
    den = acc_ref[0, 0:10000, 64:65] + acc_ref[1, 0:10000, 64:65] + wself
    o = num / den + b2_ref[...]
    m = jnp.max(o, axis=1, keepdims=True)
    lse = jnp.log(jnp.sum(jnp.exp(o - m), axis=1, keepdims=True)) + m
    out_ref[...] = o - lse


def kernel(x, edge_index, W1, att_src1, att_dst1, b1, W2, att_src2, att_dst2, b2):
    f32 = jnp.float32
    eye_h = jnp.eye(_HEADS, dtype=f32)
    t_src = (eye_h[:, None, :] * att_src1.astype(f32).T[None, :, :]).reshape(_D1, _HEADS)
    t_dst = (eye_h[:, None, :] * att_dst1.astype(f32).T[None, :, :]).reshape(_D1, _HEADS)
    aproj = jnp.concatenate([t_src, t_dst], axis=1)
    r_mat = jnp.kron(jnp.eye(_HEADS, dtype=f32), jnp.ones((1, _HID), f32))
    a2 = jnp.concatenate([att_src2.astype(f32).T, att_dst2.astype(f32).T], axis=1)
    esrc = edge_index[0].astype(jnp.int32)
    edst = edge_index[1].astype(jnp.int32)
    zeros = jnp.zeros((_NPAD, 128), f32)

    ts1, td1 = pl.pallas_call(
        _tc_stage_a,
        out_shape=[jax.ShapeDtypeStruct((2 * _N, 128), f32),
                   jax.ShapeDtypeStruct((2 * _N, 128), f32)],
    )(x.astype(f32), W1.astype(f32), aproj)

    acc1 = _sc_call1()(ts1, td1, esrc, edst, zeros)

    ts2, td2 = pl.pallas_call(
        _tc_stage_b,
        out_shape=[jax.ShapeDtypeStruct((_N, 128), f32),
                   jax.ShapeDtypeStruct((_N, 128), f32)],
    )(acc1, ts1, b1.astype(f32).reshape(1, _D1), W2.astype(f32), r_mat, a2)

    acc2 = _sc_call2()(ts2, td2, esrc, edst, zeros)

    out = pl.pallas_call(
        _tc_stage_c,
        out_shape=jax.ShapeDtypeStruct((_N, _D_OUT), f32),
    )(acc2, ts2, b2.astype(f32).reshape(1, _D_OUT))
    return out


# speedup vs baseline: 67.0658x; 1.8683x over previous
import jax
import jax.numpy as jnp
from jax import lax
from jax.experimental import pallas as pl
from jax.experimental.pallas import tpu as pltpu
from jax.experimental.pallas import tpu_sc as plsc

_N = 10000
_E = 320000
_D_IN = 128
_HID = 16
_HEADS = 8
_D1 = _HEADS * _HID
_D_OUT = 64

_NCORES = 2
_NSUB = 16
_NPAD = 10240
_ROWS_PER_TILE = _NPAD // _NSUB

_B = 80


def _leaky(v):
    return jnp.where(v >= 0, v, 0.2 * v)


def _edge_pass(ts_hbm, td_hbm, esrc_hbm, edst_hbm, z_hbm, acc_out,
               bufs, acc_sh, base, chunks, use_goff, n_mul, w_col, n_wlanes):
    cid = lax.axis_index("c")
    sid = lax.axis_index("s")
    r0 = sid * _ROWS_PER_TILE
    pltpu.sync_copy(z_hbm.at[pl.ds(r0, _ROWS_PER_TILE), :],
                    acc_sh.at[pl.ds(r0, _ROWS_PER_TILE), :])
    plsc.subcore_barrier()

    lane = lax.iota(jnp.int32, 16)
    goff = jnp.full((16,), cid * _N, jnp.int32)

    def _fire_i(c, buf):
        off = base + c * _B
        pltpu.async_copy(esrc_hbm.at[pl.ds(off, _B)], buf[0], buf[6])
        pltpu.async_copy(edst_hbm.at[pl.ds(off, _B)], buf[3], buf[6])

    def _wait_i(c, buf):
        off = base + c * _B
        pltpu.make_async_copy(esrc_hbm.at[pl.ds(off, _B)], buf[0], buf[6]).wait()
        pltpu.make_async_copy(edst_hbm.at[pl.ds(off, _B)], buf[3], buf[6]).wait()

    def _stage(buf):
        srcg, dstg, dsts, dstidx = buf[0], buf[1], buf[2], buf[3]
        for g in range(_B // 16):
            sl = pl.ds(g * 16, 16)
            d = dstidx[sl]
            dsts[sl] = d
            if use_goff:
                srcg[sl] = srcg[sl] + goff
                dstg[sl] = d + goff
            else:
                dstg[sl] = d

    def _fire_g(buf):
        pltpu.async_copy(ts_hbm.at[buf[0]], buf[4], buf[7])
        pltpu.async_copy(td_hbm.at[buf[1]], buf[5], buf[7])

    def _wait_g(buf):
        pltpu.make_async_copy(ts_hbm.at[buf[0]], buf[4], buf[7]).wait()
        pltpu.make_async_copy(td_hbm.at[buf[1]], buf[5], buf[7]).wait()

    def _compute(buf):
        srow, drow = buf[4], buf[5]

        def _edge(k, cc):
            w16 = jnp.exp(_leaky(srow[k, pl.ds(64, 16)] + drow[k, pl.ds(0, 16)]))
            srow[k, pl.ds(w_col, 16)] = jnp.where(lane < n_wlanes, w16, 0.0)
            for h in range(n_mul):
                wv = jnp.full((16,), w16[h if n_wlanes > 1 else 0], jnp.float32)
                srow[k, pl.ds(h * 16, 16)] = srow[k, pl.ds(h * 16, 16)] * wv
            return cc

        lax.fori_loop(0, _B, _edge, 0, unroll=4)

    def _fire_s(buf):
        pltpu.async_copy(buf[4], acc_sh.at[buf[2]], buf[8], add=True)

    def _wait_s(buf):
        pltpu.make_async_copy(buf[4], acc_sh.at[buf[2]], buf[8]).wait()

    def _phase(c, buf, obuf):
        _wait_g(buf)

        @pl.when(c >= 1)
        def _():
            _wait_s(obuf)

        @pl.when(c + 1 < chunks)
        def _():
            _wait_i(c + 1, obuf)
            _stage(obuf)
            _fire_g(obuf)

        @pl.when(c + 2 < chunks)
        def _():
            _fire_i(c + 2, buf)

        _compute(buf)
        _fire_s(buf)

    _fire_i(0, bufs[0])
    _wait_i(0, bufs[0])
    _stage(bufs[0])
    _fire_g(bufs[0])
    _fire_i(1, bufs[1])

    def _pair(j, carry):
        _phase(2 * j, bufs[0], bufs[1])
        _phase(2 * j + 1, bufs[1], bufs[0])
        return carry

    lax.fori_loop(0, chunks // 2, _pair, 0)
    if chunks % 2 == 1:
        _phase(chunks - 1, bufs[0], bufs[1])
    _wait_s(bufs[(chunks - 1) % 2])

    plsc.subcore_barrier()
    pltpu.sync_copy(acc_sh.at[pl.ds(r0, _ROWS_PER_TILE), :],
                    acc_out.at[cid, pl.ds(r0, _ROWS_PER_TILE), :])


def _sc_edges1(ts_hbm, td_hbm, esrc_hbm, edst_hbm, z_hbm, acc_out,
               srcg0, dstg0, dsts0, dstidx0, srow0, drow0,
               srcg1, dstg1, dsts1, dstidx1, srow1, drow1,
               acc_sh, sem_i0, sem_g0, sem_s0, sem_i1, sem_g1, sem_s1):
    sid = lax.axis_index("s")
    epw = _E // _NSUB
    bufs = [(srcg0, dstg0, dsts0, dstidx0, srow0, drow0, sem_i0, sem_g0, sem_s0),
            (srcg1, dstg1, dsts1, dstidx1, srow1, drow1, sem_i1, sem_g1, sem_s1)]
    _edge_pass(ts_hbm, td_hbm, esrc_hbm, edst_hbm, z_hbm, acc_out,
               bufs, acc_sh, sid * epw, epw // _B,
               use_goff=True, n_mul=4, w_col=64, n_wlanes=4)


def _sc_edges2(ts_hbm, td_hbm, esrc_hbm, edst_hbm, z_hbm, acc_out,
               srcg0, dstg0, dsts0, dstidx0, srow0, drow0,
               srcg1, dstg1, dsts1, dstidx1, srow1, drow1,
               acc_sh, sem_i0, sem_g0, sem_s0, sem_i1, sem_g1, sem_s1):
    cid = lax.axis_index("c")
    sid = lax.axis_index("s")
    epw = _E // (_NCORES * _NSUB)
    bufs = [(srcg0, dstg0, dsts0, dstidx0, srow0, drow0, sem_i0, sem_g0, sem_s0),
            (srcg1, dstg1, dsts1, dstidx1, srow1, drow1, sem_i1, sem_g1, sem_s1)]
    _edge_pass(ts_hbm, td_hbm, esrc_hbm, edst_hbm, z_hbm, acc_out,
               bufs, acc_sh, (cid * _NSUB + sid) * epw, epw // _B,
               use_goff=False, n_mul=4, w_col=64, n_wlanes=1)


def _make_sc_call(body):
    mesh = plsc.VectorSubcoreMesh(core_axis_name="c", subcore_axis_name="s")
    buf = []
    for _ in range(2):
        buf += [
            pltpu.VMEM((_B,), jnp.int32),
            pltpu.VMEM((_B,), jnp.int32),
            pltpu.VMEM((_B,), jnp.int32),
            pltpu.VMEM((_B,), jnp.int32),
            pltpu.VMEM((_B, 128), jnp.float32),
            pltpu.VMEM((_B, 128), jnp.float32),
        ]
    return pl.kernel(
        body,
        out_type=jax.ShapeDtypeStruct((_NCORES, _NPAD, 128), jnp.float32),
        mesh=mesh,
        scratch_types=[
            *buf,
            pltpu.VMEM_SHARED((_NPAD, 128), jnp.float32),
            pltpu.SemaphoreType.DMA,
            pltpu.SemaphoreType.DMA,
            pltpu.SemaphoreType.DMA,
            pltpu.SemaphoreType.DMA,
            pltpu.SemaphoreType.DMA,
            pltpu.SemaphoreType.DMA,
        ],
    )


def _tc_stage_a(x_ref, w1_ref, aproj_ref, ts_out, td_out):
    h = jnp.dot(x_ref[...], w1_ref[...], preferred_element_type=jnp.float32)
    att = jnp.dot(h, aproj_ref[...], preferred_element_type=jnp.float32)
    n = h.shape[0]
    z56 = jnp.zeros((n, 56), jnp.float32)
    z124 = jnp.zeros((n, 124), jnp.float32)
    ts_out[0:10000, :] = jnp.concatenate(
        [h[:, 0:64], att[:, 0:4], att[:, 8:12], z56], axis=1)
    ts_out[10000:20000, :] = jnp.concatenate(
        [h[:, 64:128], att[:, 4:8], att[:, 12:16], z56], axis=1)
    td_out[0:10000, :] = jnp.concatenate([att[:, 8:12], z124], axis=1)
    td_out[10000:20000, :] = jnp.concatenate([att[:, 12:16], z124], axis=1)


def _tc_stage_b(acc_ref, ts1_ref, b1_ref, w2_ref, r_ref, a2_ref, ts_out, td_out):
    h1 = jnp.concatenate([ts1_ref[0:10000, 0:64], ts1_ref[10000:20000, 0:64]], axis=1)
    as1 = jnp.concatenate([ts1_ref[0:10000, 64:68], ts1_ref[10000:20000, 64:68]], axis=1)
    ad1 = jnp.concatenate([ts1_ref[0:10000, 68:72], ts1_ref[10000:20000, 68:72]], axis=1)
    wself = jnp.exp(_leaky(as1 + ad1))
    num = jnp.concatenate([acc_ref[0, 0:10000, 0:64], acc_ref[1, 0:10000, 0:64]], axis=1)
    den = jnp.concatenate([acc_ref[0, 0:10000, 64:68], acc_ref[1, 0:10000, 64:68]], axis=1)
    den = den + wself
    wexp = jnp.dot(wself, r_ref[...], preferred_element_type=jnp.float32)
    dexp = jnp.dot(den, r_ref[...], preferred_element_type=jnp.float32)
    num = num + h1 * wexp
    z = num / dexp + b1_ref[...]
    z = jnp.where(z > 0, z, jnp.exp(jnp.minimum(z, 0.0)) - 1.0)
    h2 = jnp.dot(z, w2_ref[...], preferred_element_type=jnp.float32)
    att2 = jnp.dot(h2, a2_ref[...], preferred_element_type=jnp.float32)
    n = h2.shape[0]
    ts_out[...] = jnp.concatenate([h2, att2, jnp.zeros((n, 62), jnp.float32)], axis=1)
    td_out[...] = jnp.concatenate([att2[:, 1:2], jnp.zeros((n, 127), jnp.float32)], axis=1)


def _tc_stage_c(acc_ref, ts2_ref, b2_ref, out_ref):
    h2 = ts2_ref[:, 0:64]
    wself = jnp.exp(_leaky(ts2_ref[:, 64:65] + ts2_ref[:, 65:66]))
    num = acc_ref[0, 0:10000, 0:64] + acc_ref[1, 0:10000, 0:64] + h2 * wself
    den = acc_ref[0, 0:10000, 64:65] + acc_ref[1, 0:10000, 64:65] + wself
    o = num / den + b2_ref[...]
    m = jnp.max(o, axis=1, keepdims=True)
    lse = jnp.log(jnp.sum(jnp.exp(o - m), axis=1, keepdims=True)) + m
    out_ref[...] = o - lse


def kernel(x, edge_index, W1, att_src1, att_dst1, b1, W2, att_src2, att_dst2, b2):
    f32 = jnp.float32
    eye_h = jnp.eye(_HEADS, dtype=f32)
    t_src = (eye_h[:, None, :] * att_src1.astype(f32).T[None, :, :]).reshape(_D1, _HEADS)
    t_dst = (eye_h[:, None, :] * att_dst1.astype(f32).T[None, :, :]).reshape(_D1, _HEADS)
    aproj = jnp.concatenate([t_src, t_dst], axis=1)
    r_mat = jnp.kron(jnp.eye(_HEADS, dtype=f32), jnp.ones((1, _HID), f32))
    a2 = jnp.concatenate([att_src2.astype(f32).T, att_dst2.astype(f32).T], axis=1)
    esrc = edge_index[0].astype(jnp.int32)
    edst = edge_index[1].astype(jnp.int32)
    zeros = jnp.zeros((_NPAD, 128), f32)

    ts1, td1 = pl.pallas_call(
        _tc_stage_a,
        out_shape=[jax.ShapeDtypeStruct((2 * _N, 128), f32),
                   jax.ShapeDtypeStruct((2 * _N, 128), f32)],
    )(x.astype(f32), W1.astype(f32), aproj)

    acc1 = _make_sc_call(_sc_edges1)(ts1, td1, esrc, edst, zeros)

    ts2, td2 = pl.pallas_call(
        _tc_stage_b,
        out_shape=[jax.ShapeDtypeStruct((_N, 128), f32),
                   jax.ShapeDtypeStruct((_N, 128), f32)],
    )(acc1, ts1, b1.astype(f32).reshape(1, _D1), W2.astype(f32), r_mat, a2)

    acc2 = _make_sc_call(_sc_edges2)(ts2, td2, esrc, edst, zeros)

    out = pl.pallas_call(
        _tc_stage_c,
        out_shape=jax.ShapeDtypeStruct((_N, _D_OUT), f32),
    )(acc2, ts2, b2.astype(f32).reshape(1, _D_OUT))
    return out


# speedup vs baseline: 73.6226x; 1.0978x over previous
import jax
import jax.numpy as jnp
from jax import lax
from jax.experimental import pallas as pl
from jax.experimental.pallas import tpu as pltpu
from jax.experimental.pallas import tpu_sc as plsc

_N = 10000
_E = 320000
_D_IN = 128
_HID = 16
_HEADS = 8
_D1 = _HEADS * _HID
_D_OUT = 64

_NCORES = 2
_NSUB = 16
_NPAD = 10240
_ROWS_PER_TILE = _NPAD // _NSUB

_B = 80


def _leaky(v):
    return jnp.maximum(v, 0.2 * v)


def _edge_pass(ts_hbm, td_hbm, esrc_hbm, edst_hbm, z_hbm, acc_out,
               bufs, acc_sh, base, chunks, use_goff, n_mul, w_col, n_wlanes):
    cid = lax.axis_index("c")
    sid = lax.axis_index("s")
    r0 = sid * _ROWS_PER_TILE
    pltpu.sync_copy(z_hbm.at[pl.ds(r0, _ROWS_PER_TILE), :],
                    acc_sh.at[pl.ds(r0, _ROWS_PER_TILE), :])
    plsc.subcore_barrier()

    lane = lax.iota(jnp.int32, 16)
    goff = jnp.full((16,), cid * _N, jnp.int32)

    def _fire_i(c, buf):
        off = base + c * _B
        pltpu.async_copy(esrc_hbm.at[pl.ds(off, _B)], buf[0], buf[6])
        pltpu.async_copy(edst_hbm.at[pl.ds(off, _B)], buf[3], buf[6])

    def _wait_i(c, buf):
        off = base + c * _B
        pltpu.make_async_copy(esrc_hbm.at[pl.ds(off, _B)], buf[0], buf[6]).wait()
        pltpu.make_async_copy(edst_hbm.at[pl.ds(off, _B)], buf[3], buf[6]).wait()

    def _stage(buf):
        srcg, dstg, dsts_a, dsts_b, dstidx = buf[0], buf[1], buf[2], buf[9], buf[3]
        for g in range(_B // 16):
            sl = pl.ds(g * 16, 16)
            d = dstidx[sl]
            if g < 3:
                dsts_a[pl.ds(g * 16, 16)] = d
            else:
                dsts_b[pl.ds((g - 3) * 16, 16)] = d
            if use_goff:
                srcg[sl] = srcg[sl] + goff
                dstg[sl] = d + goff
            else:
                dstg[sl] = d

    def _fire_g(buf):
        pltpu.async_copy(ts_hbm.at[buf[0]], buf[4], buf[7])
        pltpu.async_copy(td_hbm.at[buf[1]], buf[5], buf[7])

    def _wait_g(buf):
        pltpu.make_async_copy(ts_hbm.at[buf[0]], buf[4], buf[7]).wait()
        pltpu.make_async_copy(td_hbm.at[buf[1]], buf[5], buf[7]).wait()

    def _edge_fn(srow, drow):
        def _edge(k, cc):
            w16 = jnp.exp(_leaky(srow[k, pl.ds(64, 16)] + drow[k, pl.ds(0, 16)]))
            srow[k, pl.ds(w_col, 16)] = jnp.where(lane < n_wlanes, w16, 0.0)
            for h in range(n_mul):
                wv = jnp.full((16,), w16[h if n_wlanes > 1 else 0], jnp.float32)
                srow[k, pl.ds(h * 16, 16)] = srow[k, pl.ds(h * 16, 16)] * wv
            return cc
        return _edge

    def _compute_fire_s(buf):
        srow, drow = buf[4], buf[5]
        edge = _edge_fn(srow, drow)
        lax.fori_loop(0, 48, edge, 0, unroll=8)
        pltpu.async_copy(srow.at[pl.ds(0, 48)], acc_sh.at[buf[2]], buf[8], add=True)
        lax.fori_loop(48, _B, edge, 0, unroll=8)
        pltpu.async_copy(srow.at[pl.ds(48, _B - 48)], acc_sh.at[buf[9]], buf[8], add=True)

    def _wait_s(buf):
        pltpu.make_async_copy(buf[4].at[pl.ds(0, 48)], acc_sh.at[buf[2]], buf[8]).wait()
        pltpu.make_async_copy(buf[4].at[pl.ds(48, _B - 48)], acc_sh.at[buf[9]], buf[8]).wait()

    def _phase(c, buf, obuf):
        _wait_g(buf)

        @pl.when(c >= 1)
        def _():
            _wait_s(obuf)

        @pl.when(c + 1 < chunks)
        def _():
            _wait_i(c + 1, obuf)
            _stage(obuf)
            _fire_g(obuf)

        @pl.when(c + 2 < chunks)
        def _():
            _fire_i(c + 2, buf)

        _compute_fire_s(buf)

    _fire_i(0, bufs[0])
    _wait_i(0, bufs[0])
    _stage(bufs[0])
    _fire_g(bufs[0])
    _fire_i(1, bufs[1])

    def _pair(j, carry):
        _phase(2 * j, bufs[0], bufs[1])
        _phase(2 * j + 1, bufs[1], bufs[0])
        return carry

    lax.fori_loop(0, chunks // 2, _pair, 0)
    if chunks % 2 == 1:
        _phase(chunks - 1, bufs[0], bufs[1])
    _wait_s(bufs[(chunks - 1) % 2])

    plsc.subcore_barrier()
    pltpu.sync_copy(acc_sh.at[pl.ds(r0, _ROWS_PER_TILE), :],
                    acc_out.at[cid, pl.ds(r0, _ROWS_PER_TILE), :])


def _sc_edges1(ts_hbm, td_hbm, esrc_hbm, edst_hbm, z_hbm, acc_out,
               srcg0, dstg0, dsts0, dstb0, dstidx0, srow0, drow0,
               srcg1, dstg1, dsts1, dstb1, dstidx1, srow1, drow1,
               acc_sh, sem_i0, sem_g0, sem_s0, sem_i1, sem_g1, sem_s1):
    sid = lax.axis_index("s")
    epw = _E // _NSUB
    bufs = [(srcg0, dstg0, dsts0, dstidx0, srow0, drow0, sem_i0, sem_g0, sem_s0, dstb0),
            (srcg1, dstg1, dsts1, dstidx1, srow1, drow1, sem_i1, sem_g1, sem_s1, dstb1)]
    _edge_pass(ts_hbm, td_hbm, esrc_hbm, edst_hbm, z_hbm, acc_out,
               bufs, acc_sh, sid * epw, epw // _B,
               use_goff=True, n_mul=4, w_col=64, n_wlanes=4)


def _sc_edges2(ts_hbm, td_hbm, esrc_hbm, edst_hbm, z_hbm, acc_out,
               srcg0, dstg0, dsts0, dstb0, dstidx0, srow0, drow0,
               srcg1, dstg1, dsts1, dstb1, dstidx1, srow1, drow1,
               acc_sh, sem_i0, sem_g0, sem_s0, sem_i1, sem_g1, sem_s1):
    cid = lax.axis_index("c")
    sid = lax.axis_index("s")
    epw = _E // (_NCORES * _NSUB)
    bufs = [(srcg0, dstg0, dsts0, dstidx0, srow0, drow0, sem_i0, sem_g0, sem_s0, dstb0),
            (srcg1, dstg1, dsts1, dstidx1, srow1, drow1, sem_i1, sem_g1, sem_s1, dstb1)]
    _edge_pass(ts_hbm, td_hbm, esrc_hbm, edst_hbm, z_hbm, acc_out,
               bufs, acc_sh, (cid * _NSUB + sid) * epw, epw // _B,
               use_goff=False, n_mul=4, w_col=64, n_wlanes=1)


def _make_sc_call(body):
    mesh = plsc.VectorSubcoreMesh(core_axis_name="c", subcore_axis_name="s")
    buf = []
    for _ in range(2):
        buf += [
            pltpu.VMEM((_B,), jnp.int32),
            pltpu.VMEM((_B,), jnp.int32),
            pltpu.VMEM((48,), jnp.int32),
            pltpu.VMEM((_B - 48,), jnp.int32),
            pltpu.VMEM((_B,), jnp.int32),
            pltpu.VMEM((_B, 128), jnp.float32),
            pltpu.VMEM((_B, 128), jnp.float32),
        ]
    return pl.kernel(
        body,
        out_type=jax.ShapeDtypeStruct((_NCORES, _NPAD, 128), jnp.float32),
        mesh=mesh,
        scratch_types=[
            *buf,
            pltpu.VMEM_SHARED((_NPAD, 128), jnp.float32),
            pltpu.SemaphoreType.DMA,
            pltpu.SemaphoreType.DMA,
            pltpu.SemaphoreType.DMA,
            pltpu.SemaphoreType.DMA,
            pltpu.SemaphoreType.DMA,
            pltpu.SemaphoreType.DMA,
        ],
    )


def _tc_stage_a(x_ref, w1_ref, aproj_ref, ts_out, td_out):
    h = jnp.dot(x_ref[...], w1_ref[...], preferred_element_type=jnp.float32)
    att = jnp.dot(h, aproj_ref[...], preferred_element_type=jnp.float32)
    n = h.shape[0]
    z56 = jnp.zeros((n, 56), jnp.float32)
    z124 = jnp.zeros((n, 124), jnp.float32)
    ts_out[0:10000, :] = jnp.concatenate(
        [h[:, 0:64], att[:, 0:4], att[:, 8:12], z56], axis=1)
    ts_out[10000:20000, :] = jnp.concatenate(
        [h[:, 64:128], att[:, 4:8], att[:, 12:16], z56], axis=1)
    td_out[0:10000, :] = jnp.concatenate([att[:, 8:12], z124], axis=1)
    td_out[10000:20000, :] = jnp.concatenate([att[:, 12:16], z124], axis=1)


def _tc_stage_b(acc_ref, ts1_ref, b1_ref, w2_ref, r_ref, a2_ref, ts_out, td_out):
    h1 = jnp.concatenate([ts1_ref[0:10000, 0:64], ts1_ref[10000:20000, 0:64]], axis=1)
    as1 = jnp.concatenate([ts1_ref[0:10000, 64:68], ts1_ref[10000:20000, 64:68]], axis=1)
    ad1 = jnp.concatenate([ts1_ref[0:10000, 68:72], ts1_ref[10000:20000, 68:72]], axis=1)
    wself = jnp.exp(_leaky(as1 + ad1))
    num = jnp.concatenate([acc_ref[0, 0:10000, 0:64], acc_ref[1, 0:10000, 0:64]], axis=1)
    den = jnp.concatenate([acc_ref[0, 0:10000, 64:68], acc_ref[1, 0:10000, 64:68]], axis=1)
    den = den + wself
    wexp = jnp.dot(wself, r_ref[...], preferred_element_type=jnp.float32)
    dexp = jnp.dot(den, r_ref[...], preferred_element_type=jnp.float32)
    num = num + h1 * wexp
    z = num / dexp + b1_ref[...]
    z = jnp.where(z > 0, z, jnp.exp(jnp.minimum(z, 0.0)) - 1.0)
    h2 = jnp.dot(z, w2_ref[...], preferred_element_type=jnp.float32)
    att2 = jnp.dot(h2, a2_ref[...], preferred_element_type=jnp.float32)
    n = h2.shape[0]
    ts_out[...] = jnp.concatenate([h2, att2, jnp.zeros((n, 62), jnp.float32)], axis=1)
    td_out[...] = jnp.concatenate([att2[:, 1:2], jnp.zeros((n, 127), jnp.float32)], axis=1)


def _tc_stage_c(acc_ref, ts2_ref, b2_ref, out_ref):
    h2 = ts2_ref[:, 0:64]
    wself = jnp.exp(_leaky(ts2_ref[:, 64:65] + ts2_ref[:, 65:66]))
    num = acc_ref[0, 0:10000, 0:64] + acc_ref[1, 0:10000, 0:64] + h2 * wself
    den = acc_ref[0, 0:10000, 64:65] + acc_ref[1, 0:10000, 64:65] + wself
    o = num / den + b2_ref[...]
    m = jnp.max(o, axis=1, keepdims=True)
    lse = jnp.log(jnp.sum(jnp.exp(o - m), axis=1, keepdims=True)) + m
    out_ref[...] = o - lse


def kernel(x, edge_index, W1, att_src1, att_dst1, b1, W2, att_src2, att_dst2, b2):
    f32 = jnp.float32
    eye_h = jnp.eye(_HEADS, dtype=f32)
    t_src = (eye_h[:, None, :] * att_src1.astype(f32).T[None, :, :]).reshape(_D1, _HEADS)
    t_dst = (eye_h[:, None, :] * att_dst1.astype(f32).T[None, :, :]).reshape(_D1, _HEADS)
    aproj = jnp.concatenate([t_src, t_dst], axis=1)
    r_mat = jnp.kron(jnp.eye(_HEADS, dtype=f32), jnp.ones((1, _HID), f32))
    a2 = jnp.concatenate([att_src2.astype(f32).T, att_dst2.astype(f32).T], axis=1)
    esrc = edge_index[0].astype(jnp.int32)
    edst = edge_index[1].astype(jnp.int32)
    zeros = jnp.zeros((_NPAD, 128), f32)

    ts1, td1 = pl.pallas_call(
        _tc_stage_a,
        out_shape=[jax.ShapeDtypeStruct((2 * _N, 128), f32),
                   jax.ShapeDtypeStruct((2 * _N, 128), f32)],
    )(x.astype(f32), W1.astype(f32), aproj)

    acc1 = _make_sc_call(_sc_edges1)(ts1, td1, esrc, edst, zeros)

    ts2, td2 = pl.pallas_call(
        _tc_stage_b,
        out_shape=[jax.ShapeDtypeStruct((_N, 128), f32),
                   jax.ShapeDtypeStruct((_N, 128), f32)],
    )(acc1, ts1, b1.astype(f32).reshape(1, _D1), W2.astype(f32), r_mat, a2)

    acc2 = _make_sc_call(_sc_edges2)(ts2, td2, esrc, edst, zeros)

    out = pl.pallas_call(
        _tc_stage_c,
        out_shape=jax.ShapeDtypeStruct((_N, _D_OUT), f32),
    )(acc2, ts2, b2.astype(f32).reshape(1, _D_OUT))
    return out
